# baseline (speedup 1.0000x reference)
import jax
import jax.numpy as jnp
from jax import lax
from jax.experimental import pallas as pl
from jax.experimental.pallas import tpu as pltpu
from jax.experimental.pallas import tpu_sc as plsc

S, B, D, H, E, FF = 2048, 2, 1024, 16, 8, 2048
DH = D // H
T = S * B
CAP = 2 * T // E
NSLOT = E * CAP
BUF_ROWS = NSLOT + 256

_BM = 256
_BQ = 256


def _qkv_body(x_ref, w_ref, b_ref, o_ref):
    w = w_ref[...].astype(jnp.bfloat16)
    acc = lax.dot_general(x_ref[...], w, (((1,), (1,)), ((), ())),
                          preferred_element_type=jnp.float32)
    o_ref[...] = (acc + b_ref[...]).astype(jnp.bfloat16)


def _qkv_proj(x_bf, w, bias):
    bn = 512
    grid = ((3 * D) // bn,)
    return pl.pallas_call(
        _qkv_body,
        grid=grid,
        in_specs=[
            pl.BlockSpec((T, D), lambda j: (0, 0)),
            pl.BlockSpec((bn, D), lambda j: (j, 0)),
            pl.BlockSpec((1, bn), lambda j: (0, j)),
        ],
        out_specs=pl.BlockSpec((T, bn), lambda j: (0, j)),
        out_shape=jax.ShapeDtypeStruct((T, 3 * D), jnp.bfloat16),
        compiler_params=pltpu.CompilerParams(
            dimension_semantics=("arbitrary",)),
    )(x_bf, w, bias)


def _one_head(q, k, v):
    s = lax.dot_general(q, k, (((1,), (1,)), ((), ())),
                        preferred_element_type=jnp.float32)
    m = jnp.max(s, axis=-1, keepdims=True)
    p = jnp.exp(s - m)
    l = jnp.sum(p, axis=-1, keepdims=True)
    o = lax.dot_general(p.astype(jnp.bfloat16), v, (((1,), (0,)), ((), ())),
                        preferred_element_type=jnp.float32)
    return (o / l).astype(jnp.bfloat16)


def _attn_body(q_ref, k_ref, v_ref, o_ref):
    q = q_ref[...] * jnp.bfloat16(1.0 / (DH ** 0.5))
    k = k_ref[...]
    v = v_ref[...]
    o0 = _one_head(q[:, :DH], k[:, :DH], v[:, :DH])
    o1 = _one_head(q[:, DH:], k[:, DH:], v[:, DH:])
    o_ref[0] = jnp.concatenate([o0, o1], axis=-1)


def _attention(qkv):
    grid = (B, H // 2, S // _BQ)
    return pl.pallas_call(
        _attn_body,
        grid=grid,
        in_specs=[
            pl.BlockSpec((_BQ, 2 * DH),
                         lambda b, h2, i: (b * (S // _BQ) + i, h2)),
            pl.BlockSpec((S, 2 * DH), lambda b, h2, i: (b, H // 2 + h2)),
            pl.BlockSpec((S, 2 * DH), lambda b, h2, i: (b, H + h2)),
        ],
        out_specs=pl.BlockSpec((1, _BQ, 2 * DH), lambda b, h2, i: (b, i, h2)),
        out_shape=jax.ShapeDtypeStruct((B, S, D), jnp.bfloat16),
        compiler_params=pltpu.CompilerParams(
            dimension_semantics=("parallel", "parallel", "parallel")),
    )(qkv, qkv, qkv)


def _post_body(a_ref, w_ref, b_ref, src_ref, g1_ref, b1_ref, gw_ref,
               x_ref, lg_ref):
    w = w_ref[...].astype(jnp.bfloat16)
    y = lax.dot_general(a_ref[...], w, (((1,), (1,)), ((), ())),
                        preferred_element_type=jnp.float32)
    y = y + b_ref[...] + src_ref[...]
    mu = jnp.mean(y, axis=-1, keepdims=True)
    va = jnp.mean((y - mu) ** 2, axis=-1, keepdims=True)
    x = (y - mu) * lax.rsqrt(va + 1e-5) * g1_ref[...] + b1_ref[...]
    x_ref[...] = x
    lg_ref[...] = lax.dot_general(gw_ref[...], x, (((0,), (1,)), ((), ())),
                                  preferred_element_type=jnp.float32)


def _post_attn(attn_bf, w, bias, src, g1, b1, gw):
    grid = (T // _BM,)
    return pl.pallas_call(
        _post_body,
        grid=grid,
        in_specs=[
            pl.BlockSpec((_BM, D), lambda i: (i, 0)),
            pl.BlockSpec((D, D), lambda i: (0, 0)),
            pl.BlockSpec((1, D), lambda i: (0, 0)),
            pl.BlockSpec((_BM, D), lambda i: (i, 0)),
            pl.BlockSpec((1, D), lambda i: (0, 0)),
            pl.BlockSpec((1, D), lambda i: (0, 0)),
            pl.BlockSpec((D, E), lambda i: (0, 0)),
        ],
        out_specs=[
            pl.BlockSpec((_BM, D), lambda i: (i, 0)),
            pl.BlockSpec((E, _BM), lambda i: (0, i)),
        ],
        out_shape=[
            jax.ShapeDtypeStruct((T, D), jnp.float32),
            jax.ShapeDtypeStruct((E, T), jnp.float32),
        ],
        compiler_params=pltpu.CompilerParams(
            dimension_semantics=("arbitrary",)),
    )(attn_bf, w, bias, src, g1, b1, gw)


def _route_scatter_body(lgT_hbm, xf_hbm, buf_hbm, destg_hbm, gate_hbm,
                        lg_v, dest_v, destg_v, gate_v, rows_v, sem):
    cid = lax.axis_index("c")
    sid = lax.axis_index("s")
    wid = sid * 2 + cid
    tbase = wid * _TPW
    pltpu.sync_copy(lgT_hbm, lg_v)

    if True:
        def step(i, counts):
            base = i * 16
            best_v = lg_v[0, pl.ds(base, 16)]
            best_i = jnp.zeros((16,), jnp.int32)
            for e in range(1, E):
                v = lg_v[e, pl.ds(base, 16)]
                m = v > best_v
                best_v = jnp.where(m, v, best_v)
                best_i = jnp.where(m, e, best_i)
            sumexp = jnp.zeros((16,), jnp.float32)
            for e in range(E):
                v = lg_v[e, pl.ds(base, 16)]
                sumexp = sumexp + jnp.exp(v - best_v)
            gate = 1.0 / sumexp
            loc = jnp.zeros((16,), jnp.int32)
            new_counts = []
            for e in range(E):
                me = (best_i == e).astype(jnp.int32)
                c = plsc.cumsum(me)
                tot = jnp.max(c, axis=0)
                pos = c - 1 + counts[e]
                loc = jnp.where(best_i == e, pos, loc)
                new_counts.append(counts[e] + tot)
            keep = loc < CAP
            dest = jnp.where(keep, best_i * CAP + loc, NSLOT)
            dest_v[pl.ds(base, 16)] = dest
            destg_v[pl.ds(base, 16)] = jnp.minimum(dest, NSLOT - 1)
            gate_v[pl.ds(base, 16)] = jnp.where(keep, gate, 0.0)
            return tuple(new_counts)

        lax.fori_loop(0, T // 16, step,
                      tuple(jnp.int32(0) for _ in range(E)))

        @pl.when(wid == 0)
        def _():
            pltpu.sync_copy(destg_v, destg_hbm)
            pltpu.sync_copy(gate_v, gate_hbm)

        for c in range(_TPW // 16):
            idx16 = dest_v[pl.ds(tbase + c * 16, 16)]
            pltpu.sync_copy(xf_hbm.at[pl.ds(tbase + c * 16, 16)], rows_v)
            pltpu.async_copy(rows_v, buf_hbm.at[idx16], sem).wait()


def _route_scatter(lgT, x):
    return pl.kernel(
        _route_scatter_body,
        out_type=(
            jax.ShapeDtypeStruct((BUF_ROWS, D), jnp.float32),
            jax.ShapeDtypeStruct((T,), jnp.int32),
            jax.ShapeDtypeStruct((T,), jnp.float32),
        ),
        mesh=plsc.VectorSubcoreMesh(core_axis_name="c", subcore_axis_name="s"),
        scratch_types=[
            pltpu.VMEM((E, T), jnp.float32),
            pltpu.VMEM((T,), jnp.int32),
            pltpu.VMEM((T,), jnp.int32),
            pltpu.VMEM((T,), jnp.float32),
            pltpu.VMEM((16, D), jnp.float32),
            pltpu.SemaphoreType.DMA,
        ],
        compiler_params=pltpu.CompilerParams(needs_layout_passes=False),
    )(lgT, x)


_NW = 32
_TPW = T // _NW
_CHUNK = 32


def _gather_body(yf_hbm, destg2_hbm, out_hbm, idx_v, rows_v, sem):
    cid = lax.axis_index("c")
    sid = lax.axis_index("s")
    wid = sid * 2 + cid
    tbase = wid * _TPW
    pltpu.sync_copy(destg2_hbm.at[pl.ds(wid * (_TPW // _CHUNK),
                                        _TPW // _CHUNK)], idx_v)
    for c in range(_TPW // _CHUNK):
        pltpu.async_copy(yf_hbm.at[idx_v.at[c]], rows_v, sem).wait()
        pltpu.sync_copy(rows_v, out_hbm.at[pl.ds(tbase + c * _CHUNK, _CHUNK)])


def _gather(yf, destg2):
    return pl.kernel(
        _gather_body,
        out_type=jax.ShapeDtypeStruct((T, D), jnp.float32),
        mesh=plsc.VectorSubcoreMesh(core_axis_name="c", subcore_axis_name="s"),
        scratch_types=[
            pltpu.VMEM((_TPW // _CHUNK, _CHUNK), jnp.int32),
            pltpu.VMEM((_CHUNK, D), jnp.float32),
            pltpu.SemaphoreType.DMA,
        ],
    )(yf, destg2)


_BF = 1024


def _ffn_body(x_ref, w1_ref, b1_ref, w2_ref, b2_ref, o_ref):
    j = pl.program_id(1)
    x = x_ref[...].astype(jnp.bfloat16)
    w1 = w1_ref[0].astype(jnp.bfloat16)
    h = lax.dot_general(x, w1, (((1,), (0,)), ((), ())),
                        preferred_element_type=jnp.float32)
    h = jnp.maximum(h + b1_ref[0], 0.0).astype(jnp.bfloat16)
    w2 = w2_ref[0].astype(jnp.bfloat16)
    y = lax.dot_general(h, w2, (((1,), (0,)), ((), ())),
                        preferred_element_type=jnp.float32)

    @pl.when(j == 0)
    def _():
        o_ref[...] = y + b2_ref[0]

    @pl.when(j > 0)
    def _():
        o_ref[...] += y


def _expert_ffn(buf, w1, b1, w2, b2):
    grid = (E, FF // _BF)
    return pl.pallas_call(
        _ffn_body,
        grid=grid,
        in_specs=[
            pl.BlockSpec((CAP, D), lambda e, j: (e, 0)),
            pl.BlockSpec((1, D, _BF), lambda e, j: (e, 0, j)),
            pl.BlockSpec((1, 1, _BF), lambda e, j: (e, 0, j)),
            pl.BlockSpec((1, _BF, D), lambda e, j: (e, j, 0)),
            pl.BlockSpec((1, 1, D), lambda e, j: (e, 0, 0)),
        ],
        out_specs=pl.BlockSpec((CAP, D), lambda e, j: (e, 0)),
        out_shape=jax.ShapeDtypeStruct((NSLOT, D), jnp.float32),
        compiler_params=pltpu.CompilerParams(
            dimension_semantics=("parallel", "arbitrary")),
    )(buf, w1, b1, w2, b2)


def _fin_body(x_ref, y_ref, g_ref, n2g_ref, n2b_ref, o_ref):
    g = g_ref[...]
    y = jnp.where(g > 0.0, y_ref[...], 0.0) * g
    z = x_ref[...] + y
    mu = jnp.mean(z, axis=-1, keepdims=True)
    va = jnp.mean((z - mu) ** 2, axis=-1, keepdims=True)
    o_ref[...] = (z - mu) * lax.rsqrt(va + 1e-5) * n2g_ref[...] + n2b_ref[...]


def _finalize(x, y, gate2d, n2g, n2b):
    grid = (T // _BM,)
    return pl.pallas_call(
        _fin_body,
        grid=grid,
        in_specs=[
            pl.BlockSpec((_BM, D), lambda i: (i, 0)),
            pl.BlockSpec((_BM, D), lambda i: (i, 0)),
            pl.BlockSpec((_BM, 1), lambda i: (i, 0)),
            pl.BlockSpec((1, D), lambda i: (0, 0)),
            pl.BlockSpec((1, D), lambda i: (0, 0)),
        ],
        out_specs=pl.BlockSpec((_BM, D), lambda i: (i, 0)),
        out_shape=jax.ShapeDtypeStruct((T, D), jnp.float32),
        compiler_params=pltpu.CompilerParams(
            dimension_semantics=("arbitrary",)),
    )(x, y, gate2d, n2g, n2b)


@jax.jit
def kernel(src, in_proj_w, in_proj_b, out_proj_w, out_proj_b,
           norm1_g, norm1_b, norm2_g, norm2_b, gate_w, w1, b1, w2, b2):
    src_f = src.transpose(1, 0, 2).reshape(T, D)
    qkv = _qkv_proj(src_f.astype(jnp.bfloat16), in_proj_w,
                    in_proj_b.reshape(1, 3 * D))
    attn = _attention(qkv).reshape(T, D)
    x, lgT = _post_attn(attn, out_proj_w, out_proj_b.reshape(1, D), src_f,
                        norm1_g.reshape(1, D), norm1_b.reshape(1, D), gate_w)
    buf, destg, gate = _route_scatter(lgT, x)
    yf = _expert_ffn(buf, w1, b1.reshape(E, 1, FF), w2, b2.reshape(E, 1, D))
    y = _gather(yf, destg.reshape(T // _CHUNK, _CHUNK))
    out = _finalize(x, y, gate.reshape(T, 1),
                    norm2_g.reshape(1, D), norm2_b.reshape(1, D))
    return out.reshape(B, S, D).transpose(1, 0, 2)


# speedup vs baseline: 1.0107x; 1.0107x over previous
import jax
import jax.numpy as jnp
from jax import lax
from jax.experimental import pallas as pl
from jax.experimental.pallas import tpu as pltpu
from jax.experimental.pallas import tpu_sc as plsc

S, B, D, H, E, FF = 2048, 2, 1024, 16, 8, 2048
DH = D // H
T = S * B
CAP = 2 * T // E
NSLOT = E * CAP
BUF_ROWS = NSLOT + 256

_BM = 256
_BQ = 256


def _qkv_body(x_ref, w_ref, b_ref, o_ref):
    w = w_ref[...].astype(jnp.bfloat16)
    acc = lax.dot_general(x_ref[...], w, (((1,), (1,)), ((), ())),
                          preferred_element_type=jnp.float32)
    o_ref[...] = (acc + b_ref[...]).astype(jnp.bfloat16)


def _qkv_proj(x_bf, w, bias):
    bn = 512
    grid = ((3 * D) // bn,)
    return pl.pallas_call(
        _qkv_body,
        grid=grid,
        in_specs=[
            pl.BlockSpec((T, D), lambda j: (0, 0)),
            pl.BlockSpec((bn, D), lambda j: (j, 0)),
            pl.BlockSpec((1, bn), lambda j: (0, j)),
        ],
        out_specs=pl.BlockSpec((T, bn), lambda j: (0, j)),
        out_shape=jax.ShapeDtypeStruct((T, 3 * D), jnp.bfloat16),
        compiler_params=pltpu.CompilerParams(
            dimension_semantics=("arbitrary",)),
    )(x_bf, w, bias)


def _one_head(q, k, v):
    s = lax.dot_general(q, k, (((1,), (1,)), ((), ())),
                        preferred_element_type=jnp.float32)
    m = jnp.max(s, axis=-1, keepdims=True)
    p = jnp.exp(s - m)
    l = jnp.sum(p, axis=-1, keepdims=True)
    o = lax.dot_general(p.astype(jnp.bfloat16), v, (((1,), (0,)), ((), ())),
                        preferred_element_type=jnp.float32)
    return (o / l).astype(jnp.bfloat16)


def _attn_body(q_ref, k_ref, v_ref, o_ref):
    q = q_ref[...] * jnp.bfloat16(1.0 / (DH ** 0.5))
    k = k_ref[...]
    v = v_ref[...]
    o0 = _one_head(q[:, :DH], k[:, :DH], v[:, :DH])
    o1 = _one_head(q[:, DH:], k[:, DH:], v[:, DH:])
    o_ref[0] = jnp.concatenate([o0, o1], axis=-1)


def _attention(qkv):
    grid = (B, H // 2, S // _BQ)
    return pl.pallas_call(
        _attn_body,
        grid=grid,
        in_specs=[
            pl.BlockSpec((_BQ, 2 * DH),
                         lambda b, h2, i: (b * (S // _BQ) + i, h2)),
            pl.BlockSpec((S, 2 * DH), lambda b, h2, i: (b, H // 2 + h2)),
            pl.BlockSpec((S, 2 * DH), lambda b, h2, i: (b, H + h2)),
        ],
        out_specs=pl.BlockSpec((1, _BQ, 2 * DH), lambda b, h2, i: (b, i, h2)),
        out_shape=jax.ShapeDtypeStruct((B, S, D), jnp.bfloat16),
        compiler_params=pltpu.CompilerParams(
            dimension_semantics=("parallel", "parallel", "parallel")),
    )(qkv, qkv, qkv)


def _post_body(a_ref, w_ref, b_ref, src_ref, g1_ref, b1_ref, gw_ref,
               x_ref, lg_ref):
    w = w_ref[...].astype(jnp.bfloat16)
    y = lax.dot_general(a_ref[...], w, (((1,), (1,)), ((), ())),
                        preferred_element_type=jnp.float32)
    y = y + b_ref[...] + src_ref[...]
    mu = jnp.mean(y, axis=-1, keepdims=True)
    va = jnp.mean((y - mu) ** 2, axis=-1, keepdims=True)
    x = (y - mu) * lax.rsqrt(va + 1e-5) * g1_ref[...] + b1_ref[...]
    x_ref[...] = x
    lg_ref[...] = lax.dot_general(gw_ref[...], x, (((0,), (1,)), ((), ())),
                                  preferred_element_type=jnp.float32)


def _post_attn(attn_bf, w, bias, src, g1, b1, gw):
    grid = (T // _BM,)
    return pl.pallas_call(
        _post_body,
        grid=grid,
        in_specs=[
            pl.BlockSpec((_BM, D), lambda i: (i, 0)),
            pl.BlockSpec((D, D), lambda i: (0, 0)),
            pl.BlockSpec((1, D), lambda i: (0, 0)),
            pl.BlockSpec((_BM, D), lambda i: (i, 0)),
            pl.BlockSpec((1, D), lambda i: (0, 0)),
            pl.BlockSpec((1, D), lambda i: (0, 0)),
            pl.BlockSpec((D, E), lambda i: (0, 0)),
        ],
        out_specs=[
            pl.BlockSpec((_BM, D), lambda i: (i, 0)),
            pl.BlockSpec((E, _BM), lambda i: (0, i)),
        ],
        out_shape=[
            jax.ShapeDtypeStruct((T, D), jnp.float32),
            jax.ShapeDtypeStruct((E, T), jnp.float32),
        ],
        compiler_params=pltpu.CompilerParams(
            dimension_semantics=("arbitrary",)),
    )(attn_bf, w, bias, src, g1, b1, gw)


def _route_body(lgT_hbm, dest_hbm, destg_hbm, gate_hbm,
                lg_v, dest_v, destg_v, gate_v):
    cid = lax.axis_index("c")
    sid = lax.axis_index("s")

    @pl.when(jnp.logical_and(cid == 0, sid == 0))
    def _():
        pltpu.sync_copy(lgT_hbm, lg_v)

        def step(i, counts):
            base = i * 16
            best_v = lg_v[0, pl.ds(base, 16)]
            best_i = jnp.zeros((16,), jnp.int32)
            for e in range(1, E):
                v = lg_v[e, pl.ds(base, 16)]
                m = v > best_v
                best_v = jnp.where(m, v, best_v)
                best_i = jnp.where(m, e, best_i)
            sumexp = jnp.zeros((16,), jnp.float32)
            for e in range(E):
                v = lg_v[e, pl.ds(base, 16)]
                sumexp = sumexp + jnp.exp(v - best_v)
            gate = 1.0 / sumexp
            loc = jnp.zeros((16,), jnp.int32)
            new_counts = []
            for e in range(E):
                me = (best_i == e).astype(jnp.int32)
                c = plsc.cumsum(me)
                tot = jnp.max(c, axis=0)
                pos = c - 1 + counts[e]
                loc = jnp.where(best_i == e, pos, loc)
                new_counts.append(counts[e] + tot)
            keep = loc < CAP
            dest = jnp.where(keep, best_i * CAP + loc, NSLOT)
            dest_v[pl.ds(base, 16)] = dest
            destg_v[pl.ds(base, 16)] = jnp.minimum(dest, NSLOT - 1)
            gate_v[pl.ds(base, 16)] = jnp.where(keep, gate, 0.0)
            return tuple(new_counts)

        lax.fori_loop(0, T // 16, step,
                      tuple(jnp.int32(0) for _ in range(E)))
        pltpu.sync_copy(dest_v, dest_hbm)
        pltpu.sync_copy(destg_v, destg_hbm)
        pltpu.sync_copy(gate_v, gate_hbm)


def _route(lgT):
    return pl.kernel(
        _route_body,
        out_type=(
            jax.ShapeDtypeStruct((T,), jnp.int32),
            jax.ShapeDtypeStruct((T,), jnp.int32),
            jax.ShapeDtypeStruct((T,), jnp.float32),
        ),
        mesh=plsc.VectorSubcoreMesh(core_axis_name="c", subcore_axis_name="s"),
        scratch_types=[
            pltpu.VMEM((E, T), jnp.float32),
            pltpu.VMEM((T,), jnp.int32),
            pltpu.VMEM((T,), jnp.int32),
            pltpu.VMEM((T,), jnp.float32),
        ],
        compiler_params=pltpu.CompilerParams(needs_layout_passes=False),
    )(lgT)


def _scatter_body(xf_hbm, dest2_hbm, buf_hbm, idx_v, rows_v, sem):
    cid = lax.axis_index("c")
    sid = lax.axis_index("s")
    wid = sid * 2 + cid
    tbase = wid * _TPW
    pltpu.sync_copy(dest2_hbm.at[pl.ds(wid * (_TPW // _CHUNK),
                                       _TPW // _CHUNK)], idx_v)
    for c in range(_TPW // _CHUNK):
        pltpu.sync_copy(xf_hbm.at[pl.ds(tbase + c * _CHUNK, _CHUNK)], rows_v)
        pltpu.async_copy(rows_v, buf_hbm.at[idx_v.at[c]], sem).wait()


def _scatter(x, dest2):
    return pl.kernel(
        _scatter_body,
        out_type=jax.ShapeDtypeStruct((BUF_ROWS, D), jnp.float32),
        mesh=plsc.VectorSubcoreMesh(core_axis_name="c", subcore_axis_name="s"),
        scratch_types=[
            pltpu.VMEM((_TPW // _CHUNK, _CHUNK), jnp.int32),
            pltpu.VMEM((_CHUNK, D), jnp.float32),
            pltpu.SemaphoreType.DMA,
        ],
    )(x, dest2)


_NW = 32
_TPW = T // _NW
_CHUNK = 32


def _gather_body(yf_hbm, destg2_hbm, out_hbm, idx_v, rows_v, sem):
    cid = lax.axis_index("c")
    sid = lax.axis_index("s")
    wid = sid * 2 + cid
    tbase = wid * _TPW
    pltpu.sync_copy(destg2_hbm.at[pl.ds(wid * (_TPW // _CHUNK),
                                        _TPW // _CHUNK)], idx_v)
    for c in range(_TPW // _CHUNK):
        pltpu.async_copy(yf_hbm.at[idx_v.at[c]], rows_v, sem).wait()
        pltpu.sync_copy(rows_v, out_hbm.at[pl.ds(tbase + c * _CHUNK, _CHUNK)])


def _gather(yf, destg2):
    return pl.kernel(
        _gather_body,
        out_type=jax.ShapeDtypeStruct((T, D), jnp.float32),
        mesh=plsc.VectorSubcoreMesh(core_axis_name="c", subcore_axis_name="s"),
        scratch_types=[
            pltpu.VMEM((_TPW // _CHUNK, _CHUNK), jnp.int32),
            pltpu.VMEM((_CHUNK, D), jnp.float32),
            pltpu.SemaphoreType.DMA,
        ],
    )(yf, destg2)


_BF = 1024


def _ffn_body(x_ref, w1_ref, b1_ref, w2_ref, b2_ref, o_ref):
    j = pl.program_id(1)
    x = x_ref[...].astype(jnp.bfloat16)
    w1 = w1_ref[0].astype(jnp.bfloat16)
    h = lax.dot_general(x, w1, (((1,), (0,)), ((), ())),
                        preferred_element_type=jnp.float32)
    h = jnp.maximum(h + b1_ref[0], 0.0).astype(jnp.bfloat16)
    w2 = w2_ref[0].astype(jnp.bfloat16)
    y = lax.dot_general(h, w2, (((1,), (0,)), ((), ())),
                        preferred_element_type=jnp.float32)

    @pl.when(j == 0)
    def _():
        o_ref[...] = y + b2_ref[0]

    @pl.when(j > 0)
    def _():
        o_ref[...] += y


def _expert_ffn(buf, w1, b1, w2, b2):
    grid = (E, FF // _BF)
    return pl.pallas_call(
        _ffn_body,
        grid=grid,
        in_specs=[
            pl.BlockSpec((CAP, D), lambda e, j: (e, 0)),
            pl.BlockSpec((1, D, _BF), lambda e, j: (e, 0, j)),
            pl.BlockSpec((1, 1, _BF), lambda e, j: (e, 0, j)),
            pl.BlockSpec((1, _BF, D), lambda e, j: (e, j, 0)),
            pl.BlockSpec((1, 1, D), lambda e, j: (e, 0, 0)),
        ],
        out_specs=pl.BlockSpec((CAP, D), lambda e, j: (e, 0)),
        out_shape=jax.ShapeDtypeStruct((NSLOT, D), jnp.float32),
        compiler_params=pltpu.CompilerParams(
            dimension_semantics=("parallel", "arbitrary")),
    )(buf, w1, b1, w2, b2)


def _fin_body(x_ref, y_ref, g_ref, n2g_ref, n2b_ref, o_ref):
    g = g_ref[...]
    y = jnp.where(g > 0.0, y_ref[...], 0.0) * g
    z = x_ref[...] + y
    mu = jnp.mean(z, axis=-1, keepdims=True)
    va = jnp.mean((z - mu) ** 2, axis=-1, keepdims=True)
    o_ref[...] = (z - mu) * lax.rsqrt(va + 1e-5) * n2g_ref[...] + n2b_ref[...]


def _finalize(x, y, gate2d, n2g, n2b):
    grid = (T // _BM,)
    return pl.pallas_call(
        _fin_body,
        grid=grid,
        in_specs=[
            pl.BlockSpec((_BM, D), lambda i: (i, 0)),
            pl.BlockSpec((_BM, D), lambda i: (i, 0)),
            pl.BlockSpec((_BM, 1), lambda i: (i, 0)),
            pl.BlockSpec((1, D), lambda i: (0, 0)),
            pl.BlockSpec((1, D), lambda i: (0, 0)),
        ],
        out_specs=pl.BlockSpec((_BM, D), lambda i: (i, 0)),
        out_shape=jax.ShapeDtypeStruct((T, D), jnp.float32),
        compiler_params=pltpu.CompilerParams(
            dimension_semantics=("arbitrary",)),
    )(x, y, gate2d, n2g, n2b)


@jax.jit
def kernel(src, in_proj_w, in_proj_b, out_proj_w, out_proj_b,
           norm1_g, norm1_b, norm2_g, norm2_b, gate_w, w1, b1, w2, b2):
    src_f = src.transpose(1, 0, 2).reshape(T, D)
    qkv = _qkv_proj(src_f.astype(jnp.bfloat16), in_proj_w,
                    in_proj_b.reshape(1, 3 * D))
    attn = _attention(qkv).reshape(T, D)
    x, lgT = _post_attn(attn, out_proj_w, out_proj_b.reshape(1, D), src_f,
                        norm1_g.reshape(1, D), norm1_b.reshape(1, D), gate_w)
    dest, destg, gate = _route(lgT)
    buf = _scatter(x, dest.reshape(T // _CHUNK, _CHUNK))
    yf = _expert_ffn(buf, w1, b1.reshape(E, 1, FF), w2, b2.reshape(E, 1, D))
    y = _gather(yf, destg.reshape(T // _CHUNK, _CHUNK))
    out = _finalize(x, y, gate.reshape(T, 1),
                    norm2_g.reshape(1, D), norm2_b.reshape(1, D))
    return out.reshape(B, S, D).transpose(1, 0, 2)
